# baseline (speedup 1.0000x reference)
import math

import numpy as np
import jax
import jax.numpy as jnp
from jax.experimental import pallas as pl
from jax.experimental.pallas import tpu as pltpu

_ORI = 9
_CELL = 8
_BLK = 3
_EPS = 1e-5
_H = 512
_W = 512
_NC = _H // _CELL
_NB = _NC - _BLK + 1
_K = _BLK * _BLK * _ORI


def _scatter_t_const():
    s = np.zeros((_ORI * _NC, _ORI * _NC), np.float32)
    for o in range(_ORI):
        for r in range(_NC):
            s[_ORI * r + o, o * _NC + r] = 1.0
    return s


def _perm_const():
    p = np.zeros((_K, _K), np.float32)
    for j in range(_BLK):
        for i in range(_BLK):
            for o in range(_ORI):
                p[27 * j + 9 * i + o, (i * _BLK + j) * _ORI + o] = 1.0
    return p


def _hog_body(x_ref, scat_ref, perm_ref, out_ref):
    img = jnp.sqrt(x_ref[0])
    f32 = jnp.float32
    bf16 = jnp.bfloat16

    rows = jax.lax.broadcasted_iota(jnp.int32, (_H, _W), 0)
    cols = jax.lax.broadcasted_iota(jnp.int32, (_H, _W), 1)
    zrow = jnp.zeros((1, _W), f32)
    zcol = jnp.zeros((_H, 1), f32)
    up = jnp.concatenate([img[1:, :], zrow], axis=0)
    dn = jnp.concatenate([zrow, img[:-1, :]], axis=0)
    g_row = jnp.where((rows == 0) | (rows == _H - 1), 0.0, up - dn)
    lf = jnp.concatenate([img[:, 1:], zcol], axis=1)
    rt = jnp.concatenate([zcol, img[:, :-1]], axis=1)
    g_col = jnp.where((cols == 0) | (cols == _W - 1), 0.0, lf - rt)

    mag = jnp.sqrt(g_row * g_row + g_col * g_col)
    mag_bf = mag.astype(bf16)

    a = jnp.abs(g_row)
    b = jnp.where(g_row < 0.0, -g_col, g_col)
    t = jnp.where(a == 0.0, jnp.inf, b * (1.0 / a))

    jj = jax.lax.broadcasted_iota(jnp.int32, (_W, _NC), 0)
    cc = jax.lax.broadcasted_iota(jnp.int32, (_W, _NC), 1)
    pt = jnp.where((jj // _CELL) == cc, 1.0, 0.0).astype(bf16)

    rr = jax.lax.broadcasted_iota(jnp.int32, (_NC, _H), 0)
    yy = jax.lax.broadcasted_iota(jnp.int32, (_NC, _H), 1)
    pr = jnp.where((yy // _CELL) == rr, 1.0, 0.0).astype(bf16)

    def cell_pool(field_bf):
        z = jax.lax.dot(field_bf, pt, preferred_element_type=f32)
        return jax.lax.dot(pr, z.astype(bf16), preferred_element_type=f32)

    zero_bf = jnp.zeros((), bf16)
    ind = [None] + [t <= f32(1.0 / math.tan(math.radians(20.0 * k)))
                    for k in range(1, _ORI)]
    inv_area = f32(1.0 / (_CELL * _CELL))
    hist = []
    for o in range(_ORI):
        if o == 0:
            in_bin = ~ind[1]
        elif o == _ORI - 1:
            in_bin = ind[_ORI - 1]
        else:
            in_bin = ind[o] ^ ind[o + 1]
        m_o = jnp.where(in_bin, mag_bf, zero_bf)
        hist.append(cell_pool(m_o) * inv_area)

    ssq = hist[0] * hist[0]
    for o in range(1, _ORI):
        ssq = ssq + hist[o] * hist[o]
    bs = jnp.zeros((_NB, _NB), f32)
    for i in range(_BLK):
        for j in range(_BLK):
            bs = bs + ssq[i:i + _NB, j:j + _NB]
    ninv = jax.lax.rsqrt(bs + f32(_EPS * _EPS))

    ninv_t = jax.lax.dot_general(
        jnp.where(
            jax.lax.broadcasted_iota(jnp.int32, (_NB, _NB), 0)
            == jax.lax.broadcasted_iota(jnp.int32, (_NB, _NB), 1),
            1.0, 0.0),
        ninv, (((1,), (1,)), ((), ())),
        precision=jax.lax.Precision.HIGHEST,
        preferred_element_type=f32)

    hstack = jnp.concatenate([h.astype(bf16) for h in hist], axis=0)
    hitt = jax.lax.dot(scat_ref[...], hstack,
                       preferred_element_type=f32).astype(bf16)
    eye = jnp.where(
        jax.lax.broadcasted_iota(jnp.int32, (_NC, _NC), 0)
        == jax.lax.broadcasted_iota(jnp.int32, (_NC, _NC), 1),
        1.0, 0.0).astype(bf16)
    hit = jax.lax.dot_general(
        eye, hitt, (((1,), (1,)), ((), ())),
        preferred_element_type=f32).astype(bf16)

    zpad = jnp.zeros((2, _ORI * _NC), bf16)
    hs = [hit,
          jnp.concatenate([hit[1:, :], zpad[:1]], axis=0),
          jnp.concatenate([hit[2:, :], zpad], axis=0)]

    perm = perm_ref[...]

    for r in range(_NB):
        s = _ORI * r
        lhs = jnp.concatenate([h[:_NB, s:s + 27] for h in hs], axis=1)
        nb = ninv_t[:, r:r + 1].astype(bf16)
        tile = jax.lax.dot(lhs * nb, perm, preferred_element_type=f32)
        out_ref[0, r] = tile


def kernel(x):
    B = x.shape[0]
    xs = x.reshape(B, _H, _W)
    scat = jnp.asarray(_scatter_t_const(), dtype=jnp.bfloat16)
    perm = jnp.asarray(_perm_const(), dtype=jnp.bfloat16)
    nscat = _ORI * _NC
    out = pl.pallas_call(
        _hog_body,
        grid=(B,),
        in_specs=[
            pl.BlockSpec((1, _H, _W), lambda i: (i, 0, 0)),
            pl.BlockSpec((nscat, nscat), lambda i: (0, 0)),
            pl.BlockSpec((_K, _K), lambda i: (0, 0)),
        ],
        out_specs=pl.BlockSpec((1, _NB, _NB, _K), lambda i: (i, 0, 0, 0)),
        out_shape=jax.ShapeDtypeStruct((B, _NB, _NB, _K), jnp.float32),
        compiler_params=pltpu.CompilerParams(
            dimension_semantics=("arbitrary",),
        ),
    )(xs, scat, perm)
    return out.reshape(B, -1)


# speedup vs baseline: 1.1297x; 1.1297x over previous
import math

import numpy as np
import jax
import jax.numpy as jnp
from jax.experimental import pallas as pl
from jax.experimental.pallas import tpu as pltpu

_ORI = 9
_CELL = 8
_BLK = 3
_EPS = 1e-5
_H = 512
_W = 512
_NC = _H // _CELL
_NB = _NC - _BLK + 1
_K = _BLK * _BLK * _ORI


def _scatter_t_const():
    s = np.zeros((_ORI * _NC, _ORI * _NC), np.float32)
    for o in range(_ORI):
        for r in range(_NC):
            s[_ORI * r + o, o * _NC + r] = 1.0
    return s


def _perm_const():
    p = np.zeros((_K, _K), np.float32)
    for j in range(_BLK):
        for i in range(_BLK):
            for o in range(_ORI):
                p[27 * j + 9 * i + o, (i * _BLK + j) * _ORI + o] = 1.0
    return p


def _hog_body(x_ref, scat_ref, perm_ref, out_ref):
    img = jnp.sqrt(x_ref[0])
    f32 = jnp.float32
    bf16 = jnp.bfloat16

    rows = jax.lax.broadcasted_iota(jnp.int32, (_H, _W), 0)
    cols = jax.lax.broadcasted_iota(jnp.int32, (_H, _W), 1)
    zrow = jnp.zeros((1, _W), f32)
    zcol = jnp.zeros((_H, 1), f32)
    up = jnp.concatenate([img[1:, :], zrow], axis=0)
    dn = jnp.concatenate([zrow, img[:-1, :]], axis=0)
    g_row = jnp.where((rows == 0) | (rows == _H - 1), 0.0, up - dn)
    lf = jnp.concatenate([img[:, 1:], zcol], axis=1)
    rt = jnp.concatenate([zcol, img[:, :-1]], axis=1)
    g_col = jnp.where((cols == 0) | (cols == _W - 1), 0.0, lf - rt)

    mag = jnp.sqrt(g_row * g_row + g_col * g_col)
    mag_bf = mag.astype(bf16)

    a = jnp.abs(g_row)
    b = jnp.where(g_row < 0.0, -g_col, g_col)
    t = jnp.where(a == 0.0, jnp.inf, b * (1.0 / a))

    jj = jax.lax.broadcasted_iota(jnp.int32, (_W, _NC), 0)
    cc = jax.lax.broadcasted_iota(jnp.int32, (_W, _NC), 1)
    pt = jnp.where((jj // _CELL) == cc, 1.0, 0.0).astype(bf16)

    def cell_pool(field_bf):
        z = jax.lax.dot(field_bf, pt, preferred_element_type=f32)
        return jnp.sum(z.reshape(_NC, _CELL, _NC), axis=1)

    zero_bf = jnp.zeros((), bf16)
    ind = [None] + [t <= f32(1.0 / math.tan(math.radians(20.0 * k)))
                    for k in range(1, _ORI)]
    inv_area = f32(1.0 / (_CELL * _CELL))
    hist = []
    for o in range(_ORI):
        if o == 0:
            in_bin = ~ind[1]
        elif o == _ORI - 1:
            in_bin = ind[_ORI - 1]
        else:
            in_bin = ind[o] ^ ind[o + 1]
        m_o = jnp.where(in_bin, mag_bf, zero_bf)
        hist.append(cell_pool(m_o) * inv_area)

    ssq = hist[0] * hist[0]
    for o in range(1, _ORI):
        ssq = ssq + hist[o] * hist[o]
    bs = jnp.zeros((_NB, _NB), f32)
    for i in range(_BLK):
        for j in range(_BLK):
            bs = bs + ssq[i:i + _NB, j:j + _NB]
    ninv = jax.lax.rsqrt(bs + f32(_EPS * _EPS))

    ninv_t = jax.lax.dot_general(
        jnp.where(
            jax.lax.broadcasted_iota(jnp.int32, (_NB, _NB), 0)
            == jax.lax.broadcasted_iota(jnp.int32, (_NB, _NB), 1),
            1.0, 0.0),
        ninv, (((1,), (1,)), ((), ())),
        precision=jax.lax.Precision.HIGHEST,
        preferred_element_type=f32)

    hstack = jnp.concatenate([h.astype(bf16) for h in hist], axis=0)
    hitt = jax.lax.dot(scat_ref[...], hstack,
                       preferred_element_type=f32).astype(bf16)
    eye = jnp.where(
        jax.lax.broadcasted_iota(jnp.int32, (_NC, _NC), 0)
        == jax.lax.broadcasted_iota(jnp.int32, (_NC, _NC), 1),
        1.0, 0.0).astype(bf16)
    hit = jax.lax.dot_general(
        eye, hitt, (((1,), (1,)), ((), ())),
        preferred_element_type=f32).astype(bf16)

    zpad = jnp.zeros((2, _ORI * _NC), bf16)
    hs = [hit,
          jnp.concatenate([hit[1:, :], zpad[:1]], axis=0),
          jnp.concatenate([hit[2:, :], zpad], axis=0)]

    perm = perm_ref[...]

    for r in range(_NB):
        s = _ORI * r
        lhs = jnp.concatenate([h[:_NB, s:s + 27] for h in hs], axis=1)
        nb = ninv_t[:, r:r + 1].astype(bf16)
        tile = jax.lax.dot(lhs * nb, perm, preferred_element_type=f32)
        out_ref[0, r] = tile


def kernel(x):
    B = x.shape[0]
    xs = x.reshape(B, _H, _W)
    scat = jnp.asarray(_scatter_t_const(), dtype=jnp.bfloat16)
    perm = jnp.asarray(_perm_const(), dtype=jnp.bfloat16)
    nscat = _ORI * _NC
    out = pl.pallas_call(
        _hog_body,
        grid=(B,),
        in_specs=[
            pl.BlockSpec((1, _H, _W), lambda i: (i, 0, 0)),
            pl.BlockSpec((nscat, nscat), lambda i: (0, 0)),
            pl.BlockSpec((_K, _K), lambda i: (0, 0)),
        ],
        out_specs=pl.BlockSpec((1, _NB, _NB, _K), lambda i: (i, 0, 0, 0)),
        out_shape=jax.ShapeDtypeStruct((B, _NB, _NB, _K), jnp.float32),
        compiler_params=pltpu.CompilerParams(
            dimension_semantics=("arbitrary",),
        ),
    )(xs, scat, perm)
    return out.reshape(B, -1)


# speedup vs baseline: 1.1475x; 1.0157x over previous
import math

import numpy as np
import jax
import jax.numpy as jnp
from jax.experimental import pallas as pl
from jax.experimental.pallas import tpu as pltpu

_ORI = 9
_CELL = 8
_BLK = 3
_EPS = 1e-5
_H = 512
_W = 512
_NC = _H // _CELL
_NB = _NC - _BLK + 1
_K = _BLK * _BLK * _ORI
_G = 2


def _scatter_t_const():
    s = np.zeros((_ORI * _NC, _ORI * _NC), np.float32)
    for o in range(_ORI):
        for r in range(_NC):
            s[_ORI * r + o, o * _NC + r] = 1.0
    return s


def _perm_const():
    p = np.zeros((_K, _K), np.float32)
    for j in range(_BLK):
        for i in range(_BLK):
            for o in range(_ORI):
                p[27 * j + 9 * i + o, (i * _BLK + j) * _ORI + o] = 1.0
    return p


def _hog_body(x_ref, scat_ref, perm_ref, out_ref):
    for m in range(x_ref.shape[0]):
        _hog_one(x_ref, scat_ref, perm_ref, out_ref, m)


def _hog_one(x_ref, scat_ref, perm_ref, out_ref, m):
    img = jnp.sqrt(x_ref[m])
    f32 = jnp.float32
    bf16 = jnp.bfloat16

    rows = jax.lax.broadcasted_iota(jnp.int32, (_H, _W), 0)
    cols = jax.lax.broadcasted_iota(jnp.int32, (_H, _W), 1)
    zrow = jnp.zeros((1, _W), f32)
    zcol = jnp.zeros((_H, 1), f32)
    up = jnp.concatenate([img[1:, :], zrow], axis=0)
    dn = jnp.concatenate([zrow, img[:-1, :]], axis=0)
    g_row = jnp.where((rows == 0) | (rows == _H - 1), 0.0, up - dn)
    lf = jnp.concatenate([img[:, 1:], zcol], axis=1)
    rt = jnp.concatenate([zcol, img[:, :-1]], axis=1)
    g_col = jnp.where((cols == 0) | (cols == _W - 1), 0.0, lf - rt)

    mag = jnp.sqrt(g_row * g_row + g_col * g_col)
    mag_bf = mag.astype(bf16)

    a = jnp.abs(g_row)
    b = jnp.where(g_row < 0.0, -g_col, g_col)
    t = jnp.where(a == 0.0, jnp.inf, b * (1.0 / a))

    jj = jax.lax.broadcasted_iota(jnp.int32, (_W, _NC), 0)
    cc = jax.lax.broadcasted_iota(jnp.int32, (_W, _NC), 1)
    pt = jnp.where((jj // _CELL) == cc, 1.0, 0.0).astype(bf16)

    def cell_pool(field_bf):
        z = jax.lax.dot(field_bf, pt, preferred_element_type=f32)
        return jnp.sum(z.reshape(_NC, _CELL, _NC), axis=1)

    zero_bf = jnp.zeros((), bf16)
    ind = [None] + [t <= f32(1.0 / math.tan(math.radians(20.0 * k)))
                    for k in range(1, _ORI)]
    inv_area = f32(1.0 / (_CELL * _CELL))
    hist = []
    for o in range(_ORI):
        if o == 0:
            in_bin = ~ind[1]
        elif o == _ORI - 1:
            in_bin = ind[_ORI - 1]
        else:
            in_bin = ind[o] ^ ind[o + 1]
        m_o = jnp.where(in_bin, mag_bf, zero_bf)
        hist.append(cell_pool(m_o) * inv_area)

    ssq = hist[0] * hist[0]
    for o in range(1, _ORI):
        ssq = ssq + hist[o] * hist[o]
    bs = jnp.zeros((_NB, _NB), f32)
    for i in range(_BLK):
        for j in range(_BLK):
            bs = bs + ssq[i:i + _NB, j:j + _NB]
    ninv = jax.lax.rsqrt(bs + f32(_EPS * _EPS))

    ninv_t = jax.lax.dot_general(
        jnp.where(
            jax.lax.broadcasted_iota(jnp.int32, (_NB, _NB), 0)
            == jax.lax.broadcasted_iota(jnp.int32, (_NB, _NB), 1),
            1.0, 0.0),
        ninv, (((1,), (1,)), ((), ())),
        precision=jax.lax.Precision.HIGHEST,
        preferred_element_type=f32)

    hstack = jnp.concatenate([h.astype(bf16) for h in hist], axis=0)
    hitt = jax.lax.dot(scat_ref[...], hstack,
                       preferred_element_type=f32).astype(bf16)
    eye = jnp.where(
        jax.lax.broadcasted_iota(jnp.int32, (_NC, _NC), 0)
        == jax.lax.broadcasted_iota(jnp.int32, (_NC, _NC), 1),
        1.0, 0.0).astype(bf16)
    hit = jax.lax.dot_general(
        eye, hitt, (((1,), (1,)), ((), ())),
        preferred_element_type=f32).astype(bf16)

    zpad = jnp.zeros((2, _ORI * _NC), bf16)
    hs = [hit,
          jnp.concatenate([hit[1:, :], zpad[:1]], axis=0),
          jnp.concatenate([hit[2:, :], zpad], axis=0)]

    perm = perm_ref[...]

    for r in range(_NB):
        s = _ORI * r
        lhs = jnp.concatenate([h[:_NB, s:s + 27] for h in hs], axis=1)
        nb = ninv_t[:, r:r + 1].astype(bf16)
        tile = jax.lax.dot(lhs * nb, perm, preferred_element_type=f32)
        out_ref[m, r] = tile


def kernel(x):
    B = x.shape[0]
    xs = x.reshape(B, _H, _W)
    scat = jnp.asarray(_scatter_t_const(), dtype=jnp.bfloat16)
    perm = jnp.asarray(_perm_const(), dtype=jnp.bfloat16)
    nscat = _ORI * _NC
    out = pl.pallas_call(
        _hog_body,
        grid=(B // _G,),
        in_specs=[
            pl.BlockSpec((_G, _H, _W), lambda i: (i, 0, 0)),
            pl.BlockSpec((nscat, nscat), lambda i: (0, 0)),
            pl.BlockSpec((_K, _K), lambda i: (0, 0)),
        ],
        out_specs=pl.BlockSpec((_G, _NB, _NB, _K), lambda i: (i, 0, 0, 0)),
        out_shape=jax.ShapeDtypeStruct((B, _NB, _NB, _K), jnp.float32),
        compiler_params=pltpu.CompilerParams(
            dimension_semantics=("arbitrary",),
        ),
    )(xs, scat, perm)
    return out.reshape(B, -1)


# speedup vs baseline: 1.1503x; 1.0025x over previous
import math

import numpy as np
import jax
import jax.numpy as jnp
from jax.experimental import pallas as pl
from jax.experimental.pallas import tpu as pltpu

_ORI = 9
_CELL = 8
_BLK = 3
_EPS = 1e-5
_H = 512
_W = 512
_NC = _H // _CELL
_NB = _NC - _BLK + 1
_K = _BLK * _BLK * _ORI
_G = 2


def _scatter_t_const():
    s = np.zeros((_ORI * _NC, _ORI * _NC), np.float32)
    for o in range(_ORI):
        for r in range(_NC):
            s[_ORI * r + o, o * _NC + r] = 1.0
    return s


def _perm_const():
    p = np.zeros((_K, _K), np.float32)
    for j in range(_BLK):
        for i in range(_BLK):
            for o in range(_ORI):
                p[27 * j + 9 * i + o, (i * _BLK + j) * _ORI + o] = 1.0
    return p


def _hog_body(x_ref, scat_ref, perm_ref, out_ref):
    for m in range(x_ref.shape[0]):
        _hog_one(x_ref, scat_ref, perm_ref, out_ref, m)


def _hog_one(x_ref, scat_ref, perm_ref, out_ref, m):
    img = jnp.sqrt(x_ref[m])
    f32 = jnp.float32
    bf16 = jnp.bfloat16

    rows = jax.lax.broadcasted_iota(jnp.int32, (_H, _W), 0)
    cols = jax.lax.broadcasted_iota(jnp.int32, (_H, _W), 1)
    zrow = jnp.zeros((1, _W), f32)
    zcol = jnp.zeros((_H, 1), f32)
    up = jnp.concatenate([img[1:, :], zrow], axis=0)
    dn = jnp.concatenate([zrow, img[:-1, :]], axis=0)
    rb = (rows - 1).astype(jnp.uint32) >= jnp.uint32(_H - 2)
    g_row = jnp.where(rb, 0.0, up - dn)
    lf = jnp.concatenate([img[:, 1:], zcol], axis=1)
    rt = jnp.concatenate([zcol, img[:, :-1]], axis=1)
    cb = (cols - 1).astype(jnp.uint32) >= jnp.uint32(_W - 2)
    g_col = jnp.where(cb, 0.0, lf - rt)

    mag = jnp.sqrt(g_row * g_row + g_col * g_col)
    mag_bf = mag.astype(bf16)

    a = jnp.abs(g_row)
    b = jnp.where(g_row < 0.0, -g_col, g_col)
    t = jnp.where(a == 0.0, jnp.inf, b * (1.0 / a))

    jj = jax.lax.broadcasted_iota(jnp.int32, (_W, _NC), 0)
    cc = jax.lax.broadcasted_iota(jnp.int32, (_W, _NC), 1)
    pt = jnp.where((jj // _CELL) == cc, 1.0, 0.0).astype(bf16)

    def cell_pool(field_bf):
        z = jax.lax.dot(field_bf, pt, preferred_element_type=f32)
        return jnp.sum(z.reshape(_NC, _CELL, _NC), axis=1)

    zero_bf = jnp.zeros((), bf16)
    ind = [None] + [t <= f32(1.0 / math.tan(math.radians(20.0 * k)))
                    for k in range(1, _ORI)]
    inv_area = f32(1.0 / (_CELL * _CELL))
    hist = []
    for o in range(_ORI):
        if o == 0:
            in_bin = ~ind[1]
        elif o == _ORI - 1:
            in_bin = ind[_ORI - 1]
        else:
            in_bin = ind[o] ^ ind[o + 1]
        m_o = jnp.where(in_bin, mag_bf, zero_bf)
        hist.append(cell_pool(m_o) * inv_area)

    ssq = hist[0] * hist[0]
    for o in range(1, _ORI):
        ssq = ssq + hist[o] * hist[o]
    bs = jnp.zeros((_NB, _NB), f32)
    for i in range(_BLK):
        for j in range(_BLK):
            bs = bs + ssq[i:i + _NB, j:j + _NB]
    ninv = jax.lax.rsqrt(bs + f32(_EPS * _EPS))

    ninv_t = jax.lax.dot_general(
        jnp.where(
            jax.lax.broadcasted_iota(jnp.int32, (_NB, _NB), 0)
            == jax.lax.broadcasted_iota(jnp.int32, (_NB, _NB), 1),
            1.0, 0.0),
        ninv, (((1,), (1,)), ((), ())),
        precision=jax.lax.Precision.HIGHEST,
        preferred_element_type=f32)

    hstack = jnp.concatenate([h.astype(bf16) for h in hist], axis=0)
    hitt = jax.lax.dot(scat_ref[...], hstack,
                       preferred_element_type=f32).astype(bf16)
    eye = jnp.where(
        jax.lax.broadcasted_iota(jnp.int32, (_NC, _NC), 0)
        == jax.lax.broadcasted_iota(jnp.int32, (_NC, _NC), 1),
        1.0, 0.0).astype(bf16)
    hit = jax.lax.dot_general(
        eye, hitt, (((1,), (1,)), ((), ())),
        preferred_element_type=f32).astype(bf16)

    zpad = jnp.zeros((2, _ORI * _NC), bf16)
    hs = [hit,
          jnp.concatenate([hit[1:, :], zpad[:1]], axis=0),
          jnp.concatenate([hit[2:, :], zpad], axis=0)]

    perm = perm_ref[...]

    for r in range(_NB):
        s = _ORI * r
        lhs = jnp.concatenate([h[:_NB, s:s + 27] for h in hs], axis=1)
        nb = ninv_t[:, r:r + 1].astype(bf16)
        tile = jax.lax.dot(lhs * nb, perm, preferred_element_type=f32)
        out_ref[m, r] = tile


def kernel(x):
    B = x.shape[0]
    xs = x.reshape(B, _H, _W)
    scat = jnp.asarray(_scatter_t_const(), dtype=jnp.bfloat16)
    perm = jnp.asarray(_perm_const(), dtype=jnp.bfloat16)
    nscat = _ORI * _NC
    out = pl.pallas_call(
        _hog_body,
        grid=(B // _G,),
        in_specs=[
            pl.BlockSpec((_G, _H, _W), lambda i: (i, 0, 0)),
            pl.BlockSpec((nscat, nscat), lambda i: (0, 0)),
            pl.BlockSpec((_K, _K), lambda i: (0, 0)),
        ],
        out_specs=pl.BlockSpec((_G, _NB, _NB, _K), lambda i: (i, 0, 0, 0)),
        out_shape=jax.ShapeDtypeStruct((B, _NB, _NB, _K), jnp.float32),
        compiler_params=pltpu.CompilerParams(
            dimension_semantics=("arbitrary",),
        ),
    )(xs, scat, perm)
    return out.reshape(B, -1)
